# baseline (speedup 1.0000x reference)
import jax
import jax.numpy as jnp
from jax.experimental import pallas as pl


def _xw1_body(x_ref, w1_ref, o_ref):
    o_ref[...] = jnp.dot(x_ref[...], w1_ref[...],
                         preferred_element_type=jnp.float32)


def _layer1_body(adj_ref, xw1_ref, b1_ref, w2_ref, hw2_ref):
    a = adj_ref[...].astype(jnp.bfloat16)
    xw = xw1_ref[...].astype(jnp.bfloat16)
    h = jnp.dot(a, xw, preferred_element_type=jnp.float32)
    h = jnp.maximum(h + b1_ref[...], 0.0)
    hw2_ref[...] = jnp.dot(h, w2_ref[...],
                           preferred_element_type=jnp.float32)


def _layer2_body(adj_ref, hw2_ref, b2_ref, w3_ref, b3_ref, lsm_ref, out_ref):
    a = adj_ref[...].astype(jnp.bfloat16)
    hw = hw2_ref[...].astype(jnp.bfloat16)
    h2 = jnp.dot(a, hw, preferred_element_type=jnp.float32) + b2_ref[...]
    m = jnp.max(h2, axis=1, keepdims=True)
    lse = jnp.log(jnp.sum(jnp.exp(h2 - m), axis=1, keepdims=True))
    lsm_ref[...] = (h2 - m) - lse
    r = jnp.maximum(h2, 0.0)
    out_ref[...] = jnp.dot(r, w3_ref[...],
                           preferred_element_type=jnp.float32) + b3_ref[...]


def kernel(x, adj, W1, b1, W2, b2, W3, b3, encoder_type):
    n, nfeat = x.shape
    nhid = W1.shape[1]
    nclass = W2.shape[1]
    proj = W3.shape[1]
    del encoder_type

    b1r = b1.reshape(1, nhid)
    b2r = b2.reshape(1, nclass)
    b3r = b3.reshape(1, proj)

    xw1 = pl.pallas_call(
        _xw1_body,
        out_shape=jax.ShapeDtypeStruct((n, nhid), jnp.float32),
    )(x, W1)

    bm = 512
    grid = (n // bm,)

    hw2 = pl.pallas_call(
        _layer1_body,
        grid=grid,
        in_specs=[
            pl.BlockSpec((bm, n), lambda i: (i, 0)),
            pl.BlockSpec((n, nhid), lambda i: (0, 0)),
            pl.BlockSpec((1, nhid), lambda i: (0, 0)),
            pl.BlockSpec((nhid, nclass), lambda i: (0, 0)),
        ],
        out_specs=pl.BlockSpec((bm, nclass), lambda i: (i, 0)),
        out_shape=jax.ShapeDtypeStruct((n, nclass), jnp.float32),
    )(adj, xw1, b1r, W2)

    lsm, out = pl.pallas_call(
        _layer2_body,
        grid=grid,
        in_specs=[
            pl.BlockSpec((bm, n), lambda i: (i, 0)),
            pl.BlockSpec((n, nclass), lambda i: (0, 0)),
            pl.BlockSpec((1, nclass), lambda i: (0, 0)),
            pl.BlockSpec((nclass, proj), lambda i: (0, 0)),
            pl.BlockSpec((1, proj), lambda i: (0, 0)),
        ],
        out_specs=[
            pl.BlockSpec((bm, nclass), lambda i: (i, 0)),
            pl.BlockSpec((bm, proj), lambda i: (i, 0)),
        ],
        out_shape=[
            jax.ShapeDtypeStruct((n, nclass), jnp.float32),
            jax.ShapeDtypeStruct((n, proj), jnp.float32),
        ],
    )(adj, hw2, b2r, W3, b3r)

    return (lsm, out)


# speedup vs baseline: 1.1555x; 1.1555x over previous
import jax
import jax.numpy as jnp
from jax.experimental import pallas as pl
from jax.experimental.pallas import tpu as pltpu


def _gcn_body(nb, bm,
              x_ref, w1_ref, b1_ref, w2_ref, b2_ref, w3_ref, b3_ref,
              adj_ref,
              lsm_ref, out_ref,
              adj_scr, xw1_scr, hw2_scr):
    i = pl.program_id(0)

    @pl.when(i == 0)
    def _compute_xw1():
        xw1_scr[...] = jnp.dot(
            x_ref[...], w1_ref[...],
            preferred_element_type=jnp.float32).astype(jnp.bfloat16)

    @pl.when(i < nb)
    def _phase_a():
        ab = adj_ref[...].astype(jnp.bfloat16)
        adj_scr[pl.ds(i * bm, bm), :] = ab
        h = jnp.dot(ab, xw1_scr[...], preferred_element_type=jnp.float32)
        h = jnp.maximum(h + b1_ref[...], 0.0)
        hw2_scr[pl.ds(i * bm, bm), :] = jnp.dot(
            h, w2_ref[...], preferred_element_type=jnp.float32
        ).astype(jnp.bfloat16)

    @pl.when(i >= nb)
    def _phase_b():
        j = i - nb
        ab = adj_scr[pl.ds(j * bm, bm), :]
        h2 = jnp.dot(ab, hw2_scr[...],
                     preferred_element_type=jnp.float32) + b2_ref[...]
        m = jnp.max(h2, axis=1, keepdims=True)
        lse = jnp.log(jnp.sum(jnp.exp(h2 - m), axis=1, keepdims=True))
        lsm_ref[...] = (h2 - m) - lse
        r = jnp.maximum(h2, 0.0)
        out_ref[...] = jnp.dot(r, w3_ref[...],
                               preferred_element_type=jnp.float32) + b3_ref[...]


def kernel(x, adj, W1, b1, W2, b2, W3, b3, encoder_type):
    n, nfeat = x.shape
    nhid = W1.shape[1]
    nclass = W2.shape[1]
    proj = W3.shape[1]
    del encoder_type

    bm = 256
    nb = n // bm

    b1r = b1.reshape(1, nhid)
    b2r = b2.reshape(1, nclass)
    b3r = b3.reshape(1, proj)

    import functools
    body = functools.partial(_gcn_body, nb, bm)

    lsm, out = pl.pallas_call(
        body,
        grid=(2 * nb,),
        in_specs=[
            pl.BlockSpec((n, nfeat), lambda i: (0, 0)),
            pl.BlockSpec((nfeat, nhid), lambda i: (0, 0)),
            pl.BlockSpec((1, nhid), lambda i: (0, 0)),
            pl.BlockSpec((nhid, nclass), lambda i: (0, 0)),
            pl.BlockSpec((1, nclass), lambda i: (0, 0)),
            pl.BlockSpec((nclass, proj), lambda i: (0, 0)),
            pl.BlockSpec((1, proj), lambda i: (0, 0)),
            pl.BlockSpec((bm, n),
                         lambda i: (jnp.minimum(i, nb - 1), 0)),
        ],
        out_specs=[
            pl.BlockSpec((bm, nclass),
                         lambda i: (jnp.maximum(i - nb, 0), 0)),
            pl.BlockSpec((bm, proj),
                         lambda i: (jnp.maximum(i - nb, 0), 0)),
        ],
        out_shape=[
            jax.ShapeDtypeStruct((n, nclass), jnp.float32),
            jax.ShapeDtypeStruct((n, proj), jnp.float32),
        ],
        scratch_shapes=[
            pltpu.VMEM((n, n), jnp.bfloat16),
            pltpu.VMEM((n, nhid), jnp.bfloat16),
            pltpu.VMEM((n, nclass), jnp.bfloat16),
        ],
        compiler_params=pltpu.CompilerParams(
            dimension_semantics=("arbitrary",),
        ),
    )(x, W1, b1r, W2, b2r, W3, b3r, adj)

    return (lsm, out)


# speedup vs baseline: 1.3750x; 1.1899x over previous
import jax
import jax.numpy as jnp
from jax.experimental import pallas as pl
from jax.experimental.pallas import tpu as pltpu


def _gcn_body(nb, bm,
              x_ref, w1_ref, b1_ref, w2_ref, b2_ref, w3_ref, b3_ref,
              adj_ref,
              lsm_ref, out_ref,
              adj_scr, xw1_scr, hw2_scr):
    i = pl.program_id(0)

    @pl.when(i == 0)
    def _compute_xw1():
        xw1_scr[...] = jnp.dot(
            x_ref[...], w1_ref[...],
            preferred_element_type=jnp.float32).astype(jnp.bfloat16)

    @pl.when(i < nb)
    def _phase_a():
        ab = adj_ref[...].astype(jnp.bfloat16)
        adj_scr[pl.ds(i * bm, bm), :] = ab
        h = jnp.dot(ab, xw1_scr[...], preferred_element_type=jnp.float32)
        h = jnp.maximum(h + b1_ref[...], 0.0)
        hw2_scr[pl.ds(i * bm, bm), :] = jnp.dot(
            h, w2_ref[...], preferred_element_type=jnp.float32
        ).astype(jnp.bfloat16)

    @pl.when(i >= nb)
    def _phase_b():
        j = i - nb
        ab = adj_scr[pl.ds(j * bm, bm), :]
        h2 = jnp.dot(ab, hw2_scr[...],
                     preferred_element_type=jnp.float32) + b2_ref[...]
        m = jnp.max(h2, axis=1, keepdims=True)
        lse = jnp.log(jnp.sum(jnp.exp(h2 - m), axis=1, keepdims=True))
        lsm_ref[...] = (h2 - m) - lse
        r = jnp.maximum(h2, 0.0)
        out_ref[...] = jnp.dot(r, w3_ref[...],
                               preferred_element_type=jnp.float32) + b3_ref[...]


def kernel(x, adj, W1, b1, W2, b2, W3, b3, encoder_type):
    n, nfeat = x.shape
    nhid = W1.shape[1]
    nclass = W2.shape[1]
    proj = W3.shape[1]
    del encoder_type

    bm = 512
    nb = n // bm

    b1r = b1.reshape(1, nhid)
    b2r = b2.reshape(1, nclass)
    b3r = b3.reshape(1, proj)

    import functools
    body = functools.partial(_gcn_body, nb, bm)

    lsm, out = pl.pallas_call(
        body,
        grid=(2 * nb,),
        in_specs=[
            pl.BlockSpec((n, nfeat), lambda i: (0, 0)),
            pl.BlockSpec((nfeat, nhid), lambda i: (0, 0)),
            pl.BlockSpec((1, nhid), lambda i: (0, 0)),
            pl.BlockSpec((nhid, nclass), lambda i: (0, 0)),
            pl.BlockSpec((1, nclass), lambda i: (0, 0)),
            pl.BlockSpec((nclass, proj), lambda i: (0, 0)),
            pl.BlockSpec((1, proj), lambda i: (0, 0)),
            pl.BlockSpec((bm, n),
                         lambda i: (jnp.minimum(i, nb - 1), 0)),
        ],
        out_specs=[
            pl.BlockSpec((bm, nclass),
                         lambda i: (jnp.maximum(i - nb, 0), 0)),
            pl.BlockSpec((bm, proj),
                         lambda i: (jnp.maximum(i - nb, 0), 0)),
        ],
        out_shape=[
            jax.ShapeDtypeStruct((n, nclass), jnp.float32),
            jax.ShapeDtypeStruct((n, proj), jnp.float32),
        ],
        scratch_shapes=[
            pltpu.VMEM((n, n), jnp.bfloat16),
            pltpu.VMEM((n, nhid), jnp.bfloat16),
            pltpu.VMEM((n, nclass), jnp.bfloat16),
        ],
        compiler_params=pltpu.CompilerParams(
            dimension_semantics=("arbitrary",),
        ),
    )(x, W1, b1r, W2, b2r, W3, b3r, adj)

    return (lsm, out)
